# baseline (speedup 1.0000x reference)
import jax
import jax.numpy as jnp
from jax.experimental import pallas as pl
from jax.experimental.pallas import tpu as pltpu

_BB = 64


def _noise_kernel(t_ref, ab_ref, x0_ref, eps_ref, noisy_ref, eps_out_ref):
    tb = t_ref[...]
    ab_row = ab_ref[...]
    T = ab_row.shape[1]
    onehot = jax.lax.broadcasted_iota(jnp.int32, (tb.shape[0], T), 1) == tb
    abar = jnp.sum(jnp.where(onehot, ab_row, 0.0), axis=1, keepdims=True)
    a = jnp.sqrt(abar)
    b = jnp.sqrt(jnp.maximum(1.0 - abar, 0.0))
    e = eps_ref[...]
    noisy_ref[...] = a * x0_ref[...] + b * e
    eps_out_ref[...] = e


def kernel(x0, t, eps, alphabar):
    B, S, D = x0.shape
    SD = S * D
    T = alphabar.shape[0]
    x2 = x0.reshape(B, SD)
    e2 = eps.reshape(B, SD)
    t2 = t.astype(jnp.int32).reshape(B, 1)
    ab2 = alphabar.reshape(1, T)
    grid = (B // _BB,)
    noisy, eps_out = pl.pallas_call(
        _noise_kernel,
        grid=grid,
        in_specs=[
            pl.BlockSpec((_BB, 1), lambda i: (i, 0)),
            pl.BlockSpec((1, T), lambda i: (0, 0)),
            pl.BlockSpec((_BB, SD), lambda i: (i, 0)),
            pl.BlockSpec((_BB, SD), lambda i: (i, 0)),
        ],
        out_specs=[
            pl.BlockSpec((_BB, SD), lambda i: (i, 0)),
            pl.BlockSpec((_BB, SD), lambda i: (i, 0)),
        ],
        out_shape=[
            jax.ShapeDtypeStruct((B, SD), jnp.float32),
            jax.ShapeDtypeStruct((B, SD), jnp.float32),
        ],
        compiler_params=pltpu.CompilerParams(
            dimension_semantics=("arbitrary",),
        ),
    )(t2, ab2, x2, e2)
    return noisy.reshape(B, S, D), eps_out.reshape(B, S, D)


# speedup vs baseline: 1.1318x; 1.1318x over previous
import jax
import jax.numpy as jnp
from jax.experimental import pallas as pl
from jax.experimental.pallas import tpu as pltpu

_BB = 64


def _noise_kernel(t_ref, ab_ref, x0_ref, eps_ref, noisy_ref):
    tb = t_ref[...]
    ab_row = ab_ref[...]
    T = ab_row.shape[1]
    onehot = jax.lax.broadcasted_iota(jnp.int32, (tb.shape[0], T), 1) == tb
    abar = jnp.sum(jnp.where(onehot, ab_row, 0.0), axis=1, keepdims=True)
    a = jnp.sqrt(abar)
    b = jnp.sqrt(jnp.maximum(1.0 - abar, 0.0))
    noisy_ref[...] = a * x0_ref[...] + b * eps_ref[...]


def kernel(x0, t, eps, alphabar):
    B, S, D = x0.shape
    SD = S * D
    T = alphabar.shape[0]
    x2 = x0.reshape(B, SD)
    e2 = eps.reshape(B, SD)
    t2 = t.astype(jnp.int32).reshape(B, 1)
    ab2 = alphabar.reshape(1, T)
    grid = (B // _BB,)
    noisy = pl.pallas_call(
        _noise_kernel,
        grid=grid,
        in_specs=[
            pl.BlockSpec((_BB, 1), lambda i: (i, 0)),
            pl.BlockSpec((1, T), lambda i: (0, 0)),
            pl.BlockSpec((_BB, SD), lambda i: (i, 0)),
            pl.BlockSpec((_BB, SD), lambda i: (i, 0)),
        ],
        out_specs=pl.BlockSpec((_BB, SD), lambda i: (i, 0)),
        out_shape=jax.ShapeDtypeStruct((B, SD), jnp.float32),
        compiler_params=pltpu.CompilerParams(
            dimension_semantics=("parallel",),
        ),
    )(t2, ab2, x2, e2)
    return noisy.reshape(B, S, D), eps


# speedup vs baseline: 1.1356x; 1.0034x over previous
import jax
import jax.numpy as jnp
from jax.experimental import pallas as pl
from jax.experimental.pallas import tpu as pltpu

_BB = 128


def _noise_kernel(t_ref, ab_ref, x0_ref, eps_ref, noisy_ref):
    tb = t_ref[...]
    ab_row = ab_ref[...]
    T = ab_row.shape[1]
    onehot = jax.lax.broadcasted_iota(jnp.int32, (tb.shape[0], T), 1) == tb
    abar = jnp.sum(jnp.where(onehot, ab_row, 0.0), axis=1, keepdims=True)
    a = jnp.sqrt(abar)
    b = jnp.sqrt(jnp.maximum(1.0 - abar, 0.0))
    noisy_ref[...] = a * x0_ref[...] + b * eps_ref[...]


def kernel(x0, t, eps, alphabar):
    B, S, D = x0.shape
    SD = S * D
    T = alphabar.shape[0]
    x2 = x0.reshape(B, SD)
    e2 = eps.reshape(B, SD)
    t2 = t.astype(jnp.int32).reshape(B, 1)
    ab2 = alphabar.reshape(1, T)
    grid = (B // _BB,)
    noisy = pl.pallas_call(
        _noise_kernel,
        grid=grid,
        in_specs=[
            pl.BlockSpec((_BB, 1), lambda i: (i, 0)),
            pl.BlockSpec((1, T), lambda i: (0, 0)),
            pl.BlockSpec((_BB, SD), lambda i: (i, 0)),
            pl.BlockSpec((_BB, SD), lambda i: (i, 0)),
        ],
        out_specs=pl.BlockSpec((_BB, SD), lambda i: (i, 0)),
        out_shape=jax.ShapeDtypeStruct((B, SD), jnp.float32),
        compiler_params=pltpu.CompilerParams(
            dimension_semantics=("parallel",),
        ),
    )(t2, ab2, x2, e2)
    return noisy.reshape(B, S, D), eps


# speedup vs baseline: 1.1383x; 1.0023x over previous
import jax
import jax.numpy as jnp
from jax.experimental import pallas as pl
from jax.experimental.pallas import tpu as pltpu

_CB = 32
_NBUF = 6


def _noise_kernel(t_ref, ab_ref, x_hbm, e_hbm, o_hbm,
                  xb, eb, ob, sx, se, so):
    B = x_hbm.shape[0]
    nchunks = B // _CB
    ab_row = ab_ref[...]
    T = ab_row.shape[1]

    def in_copies(c, slot):
        return (
            pltpu.make_async_copy(x_hbm.at[pl.ds(c * _CB, _CB), :],
                                  xb.at[slot], sx.at[slot]),
            pltpu.make_async_copy(e_hbm.at[pl.ds(c * _CB, _CB), :],
                                  eb.at[slot], se.at[slot]),
        )

    def out_copy(c, slot):
        return pltpu.make_async_copy(ob.at[slot],
                                     o_hbm.at[pl.ds(c * _CB, _CB), :],
                                     so.at[slot])

    for s in range(_NBUF):
        cx, ce = in_copies(s, s)
        cx.start()
        ce.start()

    def body(i, _):
        slot = jax.lax.rem(i, _NBUF)
        cx, ce = in_copies(i, slot)
        cx.wait()
        ce.wait()
        tb = t_ref[pl.ds(i * _CB, _CB), :]
        onehot = jax.lax.broadcasted_iota(jnp.int32, (_CB, T), 1) == tb
        abar = jnp.sum(jnp.where(onehot, ab_row, 0.0), axis=1, keepdims=True)
        a = jnp.sqrt(abar)
        b = jnp.sqrt(jnp.maximum(1.0 - abar, 0.0))

        @pl.when(i >= _NBUF)
        def _():
            out_copy(i - _NBUF, slot).wait()

        ob[slot] = a * xb[slot] + b * eb[slot]
        out_copy(i, slot).start()

        @pl.when(i + _NBUF < nchunks)
        def _():
            nx, ne = in_copies(i + _NBUF, slot)
            nx.start()
            ne.start()

        return 0

    jax.lax.fori_loop(0, nchunks, body, 0)

    for s in range(_NBUF):
        c = nchunks - _NBUF + s
        out_copy(c, c % _NBUF).wait()


def kernel(x0, t, eps, alphabar):
    B, S, D = x0.shape
    SD = S * D
    T = alphabar.shape[0]
    x2 = x0.reshape(B, SD)
    e2 = eps.reshape(B, SD)
    t2 = t.astype(jnp.int32).reshape(B, 1)
    ab2 = alphabar.reshape(1, T)
    noisy = pl.pallas_call(
        _noise_kernel,
        in_specs=[
            pl.BlockSpec(memory_space=pltpu.VMEM),
            pl.BlockSpec(memory_space=pltpu.VMEM),
            pl.BlockSpec(memory_space=pl.ANY),
            pl.BlockSpec(memory_space=pl.ANY),
        ],
        out_specs=pl.BlockSpec(memory_space=pl.ANY),
        out_shape=jax.ShapeDtypeStruct((B, SD), jnp.float32),
        scratch_shapes=[
            pltpu.VMEM((_NBUF, _CB, SD), jnp.float32),
            pltpu.VMEM((_NBUF, _CB, SD), jnp.float32),
            pltpu.VMEM((_NBUF, _CB, SD), jnp.float32),
            pltpu.SemaphoreType.DMA((_NBUF,)),
            pltpu.SemaphoreType.DMA((_NBUF,)),
            pltpu.SemaphoreType.DMA((_NBUF,)),
        ],
    )(t2, ab2, x2, e2)
    return noisy.reshape(B, S, D), eps
